# Initial kernel scaffold; baseline (speedup 1.0000x reference)
import jax
import jax.numpy as jnp
from jax.experimental import pallas as pl


def kernel(x, edge_index, W1, W2, Wd, bd, Wo, bo):
    raise NotImplementedError("write your pallas kernel here")


# speedup vs baseline: 11.2374x; 11.2374x over previous
import functools

import jax
import jax.numpy as jnp
from jax import lax
from jax.experimental import pallas as pl
from jax.experimental.pallas import tpu as pltpu
from jax.experimental.pallas import tpu_sc as plsc

N = 10000
D = 128
E = 320000
ALPHA = 0.1

NC = 2
NS = 16
NT = NC * NS
EPT = E // NT
K = 80
NCHUNK = EPT // K
NP = 10240
RPT = NP // NS
RCH = 128
NRC = RPT // RCH
WPT = NP // NS

_MESH = plsc.VectorSubcoreMesh(core_axis_name="c", subcore_axis_name="s")


@functools.partial(
    pl.kernel,
    mesh=_MESH,
    out_type=jax.ShapeDtypeStruct((NC, NP), jnp.float32),
    scratch_types=[
        pltpu.VMEM((K,), jnp.int32),
        pltpu.VMEM((K,), jnp.float32),
        pltpu.VMEM((WPT,), jnp.float32),
        pltpu.VMEM_SHARED((NP,), jnp.float32),
    ],
)
def _deg(col_hbm, out_hbm, cidx_v, ones_v, buf_v, acc_sh):
    cid = lax.axis_index("c")
    sid = lax.axis_index("s")
    tid = cid * NS + sid

    for i in range(K // 16):
        ones_v[pl.ds(i * 16, 16)] = jnp.ones((16,), jnp.float32)

    def _zero(i, c):
        buf_v[pl.ds(i * 16, 16)] = jnp.zeros((16,), jnp.float32)
        return c

    lax.fori_loop(0, WPT // 16, _zero, 0)
    pltpu.sync_copy(buf_v, acc_sh.at[pl.ds(sid * WPT, WPT)])
    plsc.subcore_barrier()

    ebase = tid * EPT

    def _chunk(j, c):
        pltpu.sync_copy(col_hbm.at[pl.ds(ebase + j * K, K)], cidx_v)
        pltpu.sync_copy(ones_v, acc_sh.at[cidx_v], add=True)
        return c

    lax.fori_loop(0, NCHUNK, _chunk, 0)
    plsc.subcore_barrier()

    pltpu.sync_copy(acc_sh.at[pl.ds(sid * WPT, WPT)], buf_v)
    pltpu.sync_copy(buf_v, out_hbm.at[cid, pl.ds(sid * WPT, WPT)])


@functools.partial(
    pl.kernel,
    mesh=_MESH,
    out_type=jax.ShapeDtypeStruct((NC, NP, D), jnp.float32),
    scratch_types=[
        pltpu.VMEM((K,), jnp.int32),
        pltpu.VMEM((K,), jnp.int32),
        pltpu.VMEM((K, D), jnp.float32),
        pltpu.VMEM((RCH, D), jnp.float32),
        pltpu.VMEM_SHARED((NP, D), jnp.float32),
        pltpu.SemaphoreType.DMA,
    ],
)
def _scatter(xs_hbm, row_hbm, col_hbm, out_hbm, ridx_v, cidx_v, rows_v,
             tmp_v, acc_sh, sem):
    cid = lax.axis_index("c")
    sid = lax.axis_index("s")
    tid = cid * NS + sid
    r0 = sid * RPT

    def _zero(i, c):
        r = i // (D // 16)
        q = lax.rem(i, D // 16)
        tmp_v[r, pl.ds(q * 16, 16)] = jnp.zeros((16,), jnp.float32)
        return c

    lax.fori_loop(0, RCH * (D // 16), _zero, 0)
    for k in range(NRC):
        pltpu.sync_copy(tmp_v, acc_sh.at[pl.ds(r0 + k * RCH, RCH)])
    plsc.subcore_barrier()

    ebase = tid * EPT

    def _chunk(j, c):
        b = ebase + j * K
        pltpu.sync_copy(row_hbm.at[pl.ds(b, K)], ridx_v)
        pltpu.sync_copy(col_hbm.at[pl.ds(b, K)], cidx_v)
        pltpu.async_copy(xs_hbm.at[ridx_v], rows_v, sem).wait()
        pltpu.sync_copy(rows_v, acc_sh.at[cidx_v], add=True)
        return c

    lax.fori_loop(0, NCHUNK, _chunk, 0)
    plsc.subcore_barrier()

    for k in range(NRC):
        pltpu.sync_copy(acc_sh.at[pl.ds(r0 + k * RCH, RCH)], tmp_v)
        pltpu.sync_copy(tmp_v, out_hbm.at[cid, pl.ds(r0 + k * RCH, RCH)])


BN = 1000


def _row_spec(w):
    return pl.BlockSpec((BN, w), lambda i: (i, 0))


def _full_spec(h, w):
    return pl.BlockSpec((h, w), lambda i: (0, 0))


def _b0_body(d0_ref, d1_ref, x_ref, xs_ref):
    dinv = lax.rsqrt(d0_ref[...] + d1_ref[...] + 1.0)
    xs_ref[...] = x_ref[...] * dinv


def _scale_x(d0, d1, x):
    return pl.pallas_call(
        _b0_body,
        grid=(N // BN,),
        in_specs=[_row_spec(1), _row_spec(1), _row_spec(D)],
        out_specs=_row_spec(D),
        out_shape=jax.ShapeDtypeStruct((N, D), jnp.float32),
    )(d0, d1, x)


def _leaky(h):
    return jnp.where(h >= 0.0, h, 0.01 * h)


def _t1_body(p0_ref, p1_ref, xs_ref, x_ref, d0_ref, d1_ref, w_ref, out_ref):
    dinv = lax.rsqrt(d0_ref[...] + d1_ref[...] + 1.0)
    prop = dinv * (p0_ref[...] + p1_ref[...] + xs_ref[...])
    pre = (1.0 - ALPHA) * prop + ALPHA * x_ref[...]
    h = jnp.dot(pre, w_ref[...], preferred_element_type=jnp.float32,
                precision=lax.Precision.HIGHEST)
    out_ref[...] = _leaky(h) * dinv


def _layer1(p0, p1, xs, x, d0, d1, W):
    return pl.pallas_call(
        _t1_body,
        grid=(N // BN,),
        in_specs=[_row_spec(D), _row_spec(D), _row_spec(D), _row_spec(D),
                  _row_spec(1), _row_spec(1), _full_spec(D, D)],
        out_specs=_row_spec(D),
        out_shape=jax.ShapeDtypeStruct((N, D), jnp.float32),
    )(p0, p1, xs, x, d0, d1, W)


def _t2_body(p0_ref, p1_ref, xs_ref, x_ref, d0_ref, d1_ref, w2_ref,
             wd_ref, bd_ref, wo_ref, bo_ref, out_ref):
    dinv = lax.rsqrt(d0_ref[...] + d1_ref[...] + 1.0)
    prop = dinv * (p0_ref[...] + p1_ref[...] + xs_ref[...])
    pre = (1.0 - ALPHA) * prop + ALPHA * x_ref[...]
    h = _leaky(jnp.dot(pre, w2_ref[...], preferred_element_type=jnp.float32,
                       precision=lax.Precision.HIGHEST))
    t = jnp.dot(h, wd_ref[...], preferred_element_type=jnp.float32,
                precision=lax.Precision.HIGHEST) + bd_ref[...]
    out_ref[...] = jnp.dot(t, wo_ref[...], preferred_element_type=jnp.float32,
                           precision=lax.Precision.HIGHEST) + bo_ref[...]


def _layer2_dense(p0, p1, xs, x, d0, d1, W2, Wd, bd, Wo, bo):
    dd = Wd.shape[1]
    do = Wo.shape[1]
    return pl.pallas_call(
        _t2_body,
        grid=(N // BN,),
        in_specs=[_row_spec(D), _row_spec(D), _row_spec(D), _row_spec(D),
                  _row_spec(1), _row_spec(1), _full_spec(D, D),
                  _full_spec(D, dd), _full_spec(1, dd),
                  _full_spec(dd, do), _full_spec(1, do)],
        out_specs=_row_spec(do),
        out_shape=jax.ShapeDtypeStruct((N, do), jnp.float32),
    )(p0, p1, xs, x, d0, d1, W2, Wd, bd, Wo, bo)


def kernel(x, edge_index, W1, W2, Wd, bd, Wo, bo):
    row = edge_index[0]
    col = edge_index[1]

    degp = _deg(col)
    d0 = degp[0, :N].reshape(N, 1)
    d1 = degp[1, :N].reshape(N, 1)

    xs0 = _scale_x(d0, d1, x)
    p = _scatter(xs0, row, col)
    xs1 = _layer1(p[0, :N], p[1, :N], xs0, x, d0, d1, W1)
    q = _scatter(xs1, row, col)
    return _layer2_dense(q[0, :N], q[1, :N], xs1, x, d0, d1, W2, Wd,
                         bd.reshape(1, -1), Wo, bo.reshape(1, -1))
